# baseline (speedup 1.0000x reference)
import functools

import jax
import jax.numpy as jnp
from jax.experimental import pallas as pl
from jax.experimental.pallas import tpu as pltpu
from jax.experimental.pallas import tpu_sc as plsc

NUM_EXPERTS = 8
TOP_K = 2
HIDDEN = 768
INTER = 512
TOKENS = 2048
TOK_H = TOKENS // 2
BM = 256
NB_H = TOK_H * TOP_K // BM + NUM_EXPERTS - 1
PAD_H = NB_H * BM
NW = 32
TPW_S = TOK_H // NW
TPW_G = TOKENS // NW


def _sc_mesh():
    return plsc.VectorSubcoreMesh(core_axis_name="c", subcore_axis_name="s")


def _routing_body(lt_ref, d0_ref, d1_ref, w0_ref, w1_ref, bmap_ref):
    lt = lt_ref[...]
    iota_e = jax.lax.broadcasted_iota(jnp.int32, (NUM_EXPERTS, TOKENS), 0)
    col = jax.lax.broadcasted_iota(jnp.int32, (NUM_EXPERTS, TOKENS), 1)
    m0 = jnp.max(lt, axis=0, keepdims=True)
    e0 = jnp.min(jnp.where(lt >= m0, iota_e, NUM_EXPERTS), axis=0, keepdims=True)
    sel0 = iota_e == e0
    lt2 = jnp.where(sel0, -jnp.inf, lt)
    m1 = jnp.max(lt2, axis=0, keepdims=True)
    e1 = jnp.min(jnp.where(lt2 >= m1, iota_e, NUM_EXPERTS), axis=0, keepdims=True)
    sel1 = iota_e == e1

    t = jnp.exp(m1 - m0)
    w0 = 1.0 / (1.0 + t)
    w1 = 1.0 - w0

    mi = sel0.astype(jnp.int32) + sel1.astype(jnp.int32)
    c = mi
    sh = 1
    while sh < TOKENS:
        z = jnp.zeros((NUM_EXPERTS, sh), jnp.int32)
        c = c + jnp.concatenate([z, c[:, : TOKENS - sh]], axis=1)
        sh *= 2
    cmid = c[:, TOK_H - 1 : TOK_H]
    ctot = c[:, TOKENS - 1 : TOKENS]
    in_h1 = (col >= TOK_H).astype(jnp.int32)
    cex = c - mi - in_h1 * cmid

    def seg_of(counts):
        pc = ((counts + (BM - 1)) // BM) * BM
        s = pc
        shft = 1
        while shft < NUM_EXPERTS:
            z = jnp.zeros((shft, 1), jnp.int32)
            s = s + jnp.concatenate([z, s[: NUM_EXPERTS - shft]], axis=0)
            shft *= 2
        return s - pc, s

    seg0, end0 = seg_of(cmid)
    seg1, end1 = seg_of(ctot - cmid)
    seg = jnp.where(in_h1 > 0,
                    jnp.broadcast_to(seg1, (NUM_EXPERTS, TOKENS)),
                    jnp.broadcast_to(seg0, (NUM_EXPERTS, TOKENS)))
    slot = seg + cex
    dest0 = jnp.sum(jnp.where(sel0, slot, 0), axis=0, keepdims=True)
    dest1 = jnp.sum(jnp.where(sel1, slot, 0), axis=0, keepdims=True)

    biota = jax.lax.broadcasted_iota(jnp.int32, (1, 128), 1)
    rows = []
    for endblk in ((end0 // BM), (end1 // BM)):
        be = jnp.zeros((1, 128), jnp.int32)
        for e in range(NUM_EXPERTS):
            be = be + (biota >= endblk[e : e + 1, :]).astype(jnp.int32)
        rows.append(be)
    d0_ref[...] = dest0
    d1_ref[...] = dest1
    w0_ref[...] = jnp.broadcast_to(w0.T, (TOKENS, 16))
    w1_ref[...] = jnp.broadcast_to(w1.T, (TOKENS, 16))
    bmap_ref[...] = jnp.concatenate(rows, axis=0)


def _routing(router_logits):
    return pl.pallas_call(
        _routing_body,
        out_shape=(
            jax.ShapeDtypeStruct((1, TOKENS), jnp.int32),
            jax.ShapeDtypeStruct((1, TOKENS), jnp.int32),
            jax.ShapeDtypeStruct((TOKENS, 16), jnp.float32),
            jax.ShapeDtypeStruct((TOKENS, 16), jnp.float32),
            jax.ShapeDtypeStruct((2, 128), jnp.int32),
        ),
    )(router_logits.T)


def _sc_scatter_rows(hid, d0, d1, half):
    @functools.partial(
        pl.kernel,
        out_type=jax.ShapeDtypeStruct((PAD_H, HIDDEN), jnp.float32),
        mesh=_sc_mesh(),
        scratch_types=[
            pltpu.VMEM((1, TPW_S), jnp.int32),
            pltpu.VMEM((1, TPW_S), jnp.int32),
            pltpu.VMEM((TPW_S, HIDDEN), jnp.float32),
            pltpu.SemaphoreType.DMA,
            pltpu.SemaphoreType.DMA,
            pltpu.SemaphoreType.DMA,
        ],
    )
    def body(hid_hbm, d0_hbm, d1_hbm, xpad_hbm, i0_v, i1_v, rows_v, s0, s1, s2):
        wid = jax.lax.axis_index("s") * 2 + jax.lax.axis_index("c")
        base = half * TOK_H + wid * TPW_S
        c0 = pltpu.async_copy(d0_hbm.at[pl.ds(base, TPW_S)], i0_v.at[0], s0)
        c1 = pltpu.async_copy(d1_hbm.at[pl.ds(base, TPW_S)], i1_v.at[0], s1)
        c2 = pltpu.async_copy(hid_hbm.at[pl.ds(base, TPW_S)], rows_v, s2)
        c0.wait()
        c1.wait()
        c2.wait()
        c3 = pltpu.async_copy(rows_v, xpad_hbm.at[i0_v.at[0]], s0)
        c4 = pltpu.async_copy(rows_v, xpad_hbm.at[i1_v.at[0]], s1)
        c3.wait()
        c4.wait()

    return body(hid, d0, d1)


def _gmm_body(bmap_ref, x_ref, gw_ref, uw_ref, dw_ref, o_ref, *, half):
    @pl.when(bmap_ref[half, pl.program_id(0)] < NUM_EXPERTS)
    def _():
        x = x_ref[...]
        g = jnp.dot(x, gw_ref[0], preferred_element_type=jnp.float32)
        u = jnp.dot(x, uw_ref[0], preferred_element_type=jnp.float32)
        a = (g / (1.0 + jnp.exp(-g))) * u
        o_ref[...] = jnp.dot(a, dw_ref[0], preferred_element_type=jnp.float32)


def _grouped_ffn(bmap, x_pad, gate_w, up_w, down_w, half):
    def wmap(i, m):
        return (jnp.minimum(m[half, i], NUM_EXPERTS - 1), 0, 0)

    grid_spec = pltpu.PrefetchScalarGridSpec(
        num_scalar_prefetch=1,
        grid=(NB_H,),
        in_specs=[
            pl.BlockSpec((BM, HIDDEN), lambda i, m: (i, 0)),
            pl.BlockSpec((1, HIDDEN, INTER), wmap),
            pl.BlockSpec((1, HIDDEN, INTER), wmap),
            pl.BlockSpec((1, INTER, HIDDEN), wmap),
        ],
        out_specs=pl.BlockSpec((BM, HIDDEN), lambda i, m: (i, 0)),
    )
    return pl.pallas_call(
        functools.partial(_gmm_body, half=half),
        grid_spec=grid_spec,
        out_shape=jax.ShapeDtypeStruct((PAD_H, HIDDEN), jnp.float32),
    )(bmap, x_pad, gate_w, up_w, down_w)


def _sc_gather_combine(ypad_a, ypad_b, d0, d1, w0r, w1r):
    @functools.partial(
        pl.kernel,
        out_type=jax.ShapeDtypeStruct((TOKENS, HIDDEN), jnp.float32),
        mesh=_sc_mesh(),
        scratch_types=[
            pltpu.VMEM((1, TPW_G), jnp.int32),
            pltpu.VMEM((1, TPW_G), jnp.int32),
            pltpu.VMEM((TPW_G, 16), jnp.float32),
            pltpu.VMEM((TPW_G, 16), jnp.float32),
            pltpu.VMEM((TPW_G, HIDDEN), jnp.float32),
            pltpu.VMEM((TPW_G, HIDDEN), jnp.float32),
            pltpu.SemaphoreType.DMA,
            pltpu.SemaphoreType.DMA,
            pltpu.SemaphoreType.DMA,
            pltpu.SemaphoreType.DMA,
        ],
    )
    def body(ya_hbm, yb_hbm, d0_hbm, d1_hbm, w0_hbm, w1_hbm, out_hbm,
             i0_v, i1_v, w0_v, w1_v, r0_v, r1_v, s0, s1, s2, s3):
        wid = jax.lax.axis_index("s") * 2 + jax.lax.axis_index("c")
        base = wid * TPW_G
        c0 = pltpu.async_copy(d0_hbm.at[pl.ds(base, TPW_G)], i0_v.at[0], s0)
        c1 = pltpu.async_copy(d1_hbm.at[pl.ds(base, TPW_G)], i1_v.at[0], s1)
        c2 = pltpu.async_copy(w0_hbm.at[pl.ds(base, TPW_G)], w0_v, s2)
        c3 = pltpu.async_copy(w1_hbm.at[pl.ds(base, TPW_G)], w1_v, s3)
        c0.wait()
        c1.wait()

        @pl.when(wid < NW // 2)
        def _():
            pltpu.async_copy(ya_hbm.at[i0_v.at[0]], r0_v, s0).wait()
            pltpu.async_copy(ya_hbm.at[i1_v.at[0]], r1_v, s1).wait()

        @pl.when(wid >= NW // 2)
        def _():
            pltpu.async_copy(yb_hbm.at[i0_v.at[0]], r0_v, s0).wait()
            pltpu.async_copy(yb_hbm.at[i1_v.at[0]], r1_v, s1).wait()

        c2.wait()
        c3.wait()

        @pl.loop(0, TPW_G)
        def _row(j):
            wv0 = w0_v[j]
            wv1 = w1_v[j]
            for c in range(0, HIDDEN, 16):
                s = (j, pl.ds(c, 16))
                r0_v[s] = r0_v[s] * wv0 + r1_v[s] * wv1

        pltpu.sync_copy(r0_v, out_hbm.at[pl.ds(base, TPW_G)])

    return body(ypad_a, ypad_b, d0, d1, w0r, w1r)


def kernel(hidden_states, router_logits, gate_w, up_w, down_w):
    d0, d1, w0r, w1r, bmap = _routing(router_logits)
    x_a = _sc_scatter_rows(hidden_states, d0[0], d1[0], 0)
    x_b = _sc_scatter_rows(hidden_states, d0[0], d1[0], 1)
    y_a = _grouped_ffn(bmap, x_a, gate_w, up_w, down_w, 0)
    y_b = _grouped_ffn(bmap, x_b, gate_w, up_w, down_w, 1)
    return _sc_gather_combine(y_a, y_b, d0[0], d1[0], w0r, w1r)


# speedup vs baseline: 1.1870x; 1.1870x over previous
import functools

import jax
import jax.numpy as jnp
from jax.experimental import pallas as pl
from jax.experimental.pallas import tpu as pltpu
from jax.experimental.pallas import tpu_sc as plsc

NUM_EXPERTS = 8
TOP_K = 2
HIDDEN = 768
INTER = 512
TOKENS = 2048
PAIRS = TOKENS * TOP_K
BM = 256
NUM_BLOCKS = PAIRS // BM + NUM_EXPERTS - 1
PAD_ROWS = NUM_BLOCKS * BM
NW = 32
TPW = TOKENS // NW

def _sc_mesh():
    return plsc.VectorSubcoreMesh(core_axis_name="c", subcore_axis_name="s")


def _routing_body(lt_ref, d0_ref, d1_ref, w0_ref, w1_ref, bmap_ref):
    lt = lt_ref[...]
    iota_e = jax.lax.broadcasted_iota(jnp.int32, (NUM_EXPERTS, TOKENS), 0)
    m0 = jnp.max(lt, axis=0, keepdims=True)
    e0 = jnp.min(jnp.where(lt >= m0, iota_e, NUM_EXPERTS), axis=0, keepdims=True)
    sel0 = iota_e == e0
    lt2 = jnp.where(sel0, -jnp.inf, lt)
    m1 = jnp.max(lt2, axis=0, keepdims=True)
    e1 = jnp.min(jnp.where(lt2 >= m1, iota_e, NUM_EXPERTS), axis=0, keepdims=True)
    sel1 = iota_e == e1

    t = jnp.exp(m1 - m0)
    w0 = 1.0 / (1.0 + t)
    w1 = 1.0 - w0

    mi = sel0.astype(jnp.int32) + sel1.astype(jnp.int32)
    c = mi
    sh = 1
    while sh < TOKENS:
        z = jnp.zeros((NUM_EXPERTS, sh), jnp.int32)
        c = c + jnp.concatenate([z, c[:, : TOKENS - sh]], axis=1)
        sh *= 2
    counts = c[:, TOKENS - 1 : TOKENS]
    cex = c - mi
    pc = ((counts + (BM - 1)) // BM) * BM
    seg = pc
    sh = 1
    while sh < NUM_EXPERTS:
        z = jnp.zeros((sh, 1), jnp.int32)
        seg = seg + jnp.concatenate([z, seg[: NUM_EXPERTS - sh]], axis=0)
        sh *= 2
    seg = seg - pc
    slot = jnp.broadcast_to(seg, (NUM_EXPERTS, TOKENS)) + cex
    dest0 = jnp.sum(jnp.where(sel0, slot, 0), axis=0, keepdims=True)
    dest1 = jnp.sum(jnp.where(sel1, slot, 0), axis=0, keepdims=True)

    endblk = (seg + pc) // BM
    biota = jax.lax.broadcasted_iota(jnp.int32, (1, 128), 1)
    be = jnp.zeros((1, 128), jnp.int32)
    for e in range(NUM_EXPERTS):
        be = be + (biota >= endblk[e : e + 1, :]).astype(jnp.int32)
    d0_ref[...] = dest0
    d1_ref[...] = dest1
    w0_ref[...] = jnp.broadcast_to(w0.T, (TOKENS, 16))
    w1_ref[...] = jnp.broadcast_to(w1.T, (TOKENS, 16))
    bmap_ref[...] = be


def _routing(router_logits):
    return pl.pallas_call(
        _routing_body,
        out_shape=(
            jax.ShapeDtypeStruct((1, TOKENS), jnp.int32),
            jax.ShapeDtypeStruct((1, TOKENS), jnp.int32),
            jax.ShapeDtypeStruct((TOKENS, 16), jnp.float32),
            jax.ShapeDtypeStruct((TOKENS, 16), jnp.float32),
            jax.ShapeDtypeStruct((1, 128), jnp.int32),
        ),
    )(router_logits.T)


def _sc_scatter_rows(hid, d0, d1):
    @functools.partial(
        pl.kernel,
        out_type=jax.ShapeDtypeStruct((PAD_ROWS, HIDDEN), jnp.float32),
        mesh=_sc_mesh(),
        scratch_types=[
            pltpu.VMEM((1, TPW), jnp.int32),
            pltpu.VMEM((1, TPW), jnp.int32),
            pltpu.VMEM((TPW, HIDDEN), jnp.float32),
            pltpu.SemaphoreType.DMA,
            pltpu.SemaphoreType.DMA,
            pltpu.SemaphoreType.DMA,
        ],
    )
    def body(hid_hbm, d0_hbm, d1_hbm, xpad_hbm, i0_v, i1_v, rows_v, s0, s1, s2):
        wid = jax.lax.axis_index("s") * 2 + jax.lax.axis_index("c")
        base = wid * TPW
        c0 = pltpu.async_copy(d0_hbm.at[pl.ds(base, TPW)], i0_v.at[0], s0)
        c1 = pltpu.async_copy(d1_hbm.at[pl.ds(base, TPW)], i1_v.at[0], s1)
        c2 = pltpu.async_copy(hid_hbm.at[pl.ds(base, TPW)], rows_v, s2)
        c0.wait()
        c1.wait()
        c2.wait()
        c3 = pltpu.async_copy(rows_v, xpad_hbm.at[i0_v.at[0]], s0)
        c4 = pltpu.async_copy(rows_v, xpad_hbm.at[i1_v.at[0]], s1)
        c3.wait()
        c4.wait()

    return body(hid, d0, d1)


def _wmap(i, m):
    return (jnp.minimum(m[i], NUM_EXPERTS - 1), 0, 0)


def _gmm_body(bmap_ref, x_ref, gw_ref, uw_ref, dw_ref, o_ref):
    @pl.when(bmap_ref[pl.program_id(0)] < NUM_EXPERTS)
    def _():
        x = x_ref[...]
        g = jnp.dot(x, gw_ref[0], preferred_element_type=jnp.float32)
        u = jnp.dot(x, uw_ref[0], preferred_element_type=jnp.float32)
        a = (g / (1.0 + jnp.exp(-g))) * u
        o_ref[...] = jnp.dot(a, dw_ref[0], preferred_element_type=jnp.float32)


def _grouped_ffn(bmap, x_pad, gate_w, up_w, down_w):
    grid_spec = pltpu.PrefetchScalarGridSpec(
        num_scalar_prefetch=1,
        grid=(NUM_BLOCKS,),
        in_specs=[
            pl.BlockSpec((BM, HIDDEN), lambda i, m: (i, 0)),
            pl.BlockSpec((1, HIDDEN, INTER), _wmap),
            pl.BlockSpec((1, HIDDEN, INTER), _wmap),
            pl.BlockSpec((1, INTER, HIDDEN), _wmap),
        ],
        out_specs=pl.BlockSpec((BM, HIDDEN), lambda i, m: (i, 0)),
    )
    return pl.pallas_call(
        _gmm_body,
        grid_spec=grid_spec,
        out_shape=jax.ShapeDtypeStruct((PAD_ROWS, HIDDEN), jnp.float32),
    )(bmap, x_pad, gate_w, up_w, down_w)


def _sc_gather_combine(ypad, d0, d1, w0r, w1r):
    @functools.partial(
        pl.kernel,
        out_type=jax.ShapeDtypeStruct((TOKENS, HIDDEN), jnp.float32),
        mesh=_sc_mesh(),
        scratch_types=[
            pltpu.VMEM((1, TPW), jnp.int32),
            pltpu.VMEM((1, TPW), jnp.int32),
            pltpu.VMEM((TPW, 16), jnp.float32),
            pltpu.VMEM((TPW, 16), jnp.float32),
            pltpu.VMEM((TPW, HIDDEN), jnp.float32),
            pltpu.VMEM((TPW, HIDDEN), jnp.float32),
            pltpu.SemaphoreType.DMA,
            pltpu.SemaphoreType.DMA,
            pltpu.SemaphoreType.DMA,
            pltpu.SemaphoreType.DMA,
        ],
    )
    def body(ypad_hbm, d0_hbm, d1_hbm, w0_hbm, w1_hbm, out_hbm,
             i0_v, i1_v, w0_v, w1_v, r0_v, r1_v, s0, s1, s2, s3):
        wid = jax.lax.axis_index("s") * 2 + jax.lax.axis_index("c")
        base = wid * TPW
        c0 = pltpu.async_copy(d0_hbm.at[pl.ds(base, TPW)], i0_v.at[0], s0)
        c1 = pltpu.async_copy(d1_hbm.at[pl.ds(base, TPW)], i1_v.at[0], s1)
        c2 = pltpu.async_copy(w0_hbm.at[pl.ds(base, TPW)], w0_v, s2)
        c3 = pltpu.async_copy(w1_hbm.at[pl.ds(base, TPW)], w1_v, s3)
        c0.wait()
        c1.wait()
        g0 = pltpu.async_copy(ypad_hbm.at[i0_v.at[0]], r0_v, s0)
        g1 = pltpu.async_copy(ypad_hbm.at[i1_v.at[0]], r1_v, s1)
        c2.wait()
        c3.wait()
        g0.wait()
        g1.wait()

        @pl.loop(0, TPW)
        def _row(j):
            wv0 = w0_v[j]
            wv1 = w1_v[j]
            for c in range(0, HIDDEN, 16):
                s = (j, pl.ds(c, 16))
                r0_v[s] = r0_v[s] * wv0 + r1_v[s] * wv1

        pltpu.sync_copy(r0_v, out_hbm.at[pl.ds(base, TPW)])

    return body(ypad, d0, d1, w0r, w1r)


def kernel(hidden_states, router_logits, gate_w, up_w, down_w):
    d0, d1, w0r, w1r, bmap = _routing(router_logits)
    x_pad = _sc_scatter_rows(hidden_states, d0[0], d1[0])
    y_pad = _grouped_ffn(bmap[0], x_pad, gate_w, up_w, down_w)
    return _sc_gather_combine(y_pad, d0[0], d1[0], w0r, w1r)


# speedup vs baseline: 1.1929x; 1.0050x over previous
import functools

import jax
import jax.numpy as jnp
from jax.experimental import pallas as pl
from jax.experimental.pallas import tpu as pltpu
from jax.experimental.pallas import tpu_sc as plsc

NUM_EXPERTS = 8
TOP_K = 2
HIDDEN = 768
INTER = 512
TOKENS = 2048
PAIRS = TOKENS * TOP_K
BM = 256
NUM_BLOCKS = PAIRS // BM + NUM_EXPERTS - 1
PAD_ROWS = NUM_BLOCKS * BM
NW = 32
TPW = TOKENS // NW

def _sc_mesh():
    return plsc.VectorSubcoreMesh(core_axis_name="c", subcore_axis_name="s")


def _routing_body(lt_ref, d0_ref, d1_ref, w0_ref, w1_ref, bmap_ref):
    lt = lt_ref[...]
    iota_e = jax.lax.broadcasted_iota(jnp.int32, (NUM_EXPERTS, TOKENS), 0)
    m0 = jnp.max(lt, axis=0, keepdims=True)
    e0 = jnp.min(jnp.where(lt >= m0, iota_e, NUM_EXPERTS), axis=0, keepdims=True)
    sel0 = iota_e == e0
    lt2 = jnp.where(sel0, -jnp.inf, lt)
    m1 = jnp.max(lt2, axis=0, keepdims=True)
    e1 = jnp.min(jnp.where(lt2 >= m1, iota_e, NUM_EXPERTS), axis=0, keepdims=True)
    sel1 = iota_e == e1

    t = jnp.exp(m1 - m0)
    w0 = 1.0 / (1.0 + t)
    w1 = 1.0 - w0

    mi = sel0.astype(jnp.int32) + sel1.astype(jnp.int32)
    c = mi
    sh = 1
    while sh < TOKENS:
        z = jnp.zeros((NUM_EXPERTS, sh), jnp.int32)
        c = c + jnp.concatenate([z, c[:, : TOKENS - sh]], axis=1)
        sh *= 2
    counts = c[:, TOKENS - 1 : TOKENS]
    cex = c - mi
    pc = ((counts + (BM - 1)) // BM) * BM
    seg = pc
    sh = 1
    while sh < NUM_EXPERTS:
        z = jnp.zeros((sh, 1), jnp.int32)
        seg = seg + jnp.concatenate([z, seg[: NUM_EXPERTS - sh]], axis=0)
        sh *= 2
    seg = seg - pc
    slot = jnp.broadcast_to(seg, (NUM_EXPERTS, TOKENS)) + cex
    dest0 = jnp.sum(jnp.where(sel0, slot, 0), axis=0, keepdims=True)
    dest1 = jnp.sum(jnp.where(sel1, slot, 0), axis=0, keepdims=True)

    endblk = (seg + pc) // BM
    biota = jax.lax.broadcasted_iota(jnp.int32, (1, 128), 1)
    be = jnp.zeros((1, 128), jnp.int32)
    for e in range(NUM_EXPERTS):
        be = be + (biota >= endblk[e : e + 1, :]).astype(jnp.int32)
    d0_ref[...] = dest0
    d1_ref[...] = dest1
    w0_ref[...] = jnp.broadcast_to(w0.T, (TOKENS, 16))
    w1_ref[...] = jnp.broadcast_to(w1.T, (TOKENS, 16))
    bmap_ref[...] = be


def _routing(router_logits):
    return pl.pallas_call(
        _routing_body,
        out_shape=(
            jax.ShapeDtypeStruct((1, TOKENS), jnp.int32),
            jax.ShapeDtypeStruct((1, TOKENS), jnp.int32),
            jax.ShapeDtypeStruct((TOKENS, 16), jnp.float32),
            jax.ShapeDtypeStruct((TOKENS, 16), jnp.float32),
            jax.ShapeDtypeStruct((1, 128), jnp.int32),
        ),
    )(router_logits.T)


def _sc_scatter_rows(hid, d0, d1):
    @functools.partial(
        pl.kernel,
        out_type=jax.ShapeDtypeStruct((PAD_ROWS, HIDDEN), jnp.float32),
        mesh=_sc_mesh(),
        scratch_types=[
            pltpu.VMEM((1, TPW), jnp.int32),
            pltpu.VMEM((1, TPW), jnp.int32),
            pltpu.VMEM((TPW, HIDDEN), jnp.float32),
            pltpu.SemaphoreType.DMA,
            pltpu.SemaphoreType.DMA,
            pltpu.SemaphoreType.DMA,
        ],
    )
    def body(hid_hbm, d0_hbm, d1_hbm, xpad_hbm, i0_v, i1_v, rows_v, s0, s1, s2):
        wid = jax.lax.axis_index("s") * 2 + jax.lax.axis_index("c")
        base = wid * TPW
        c0 = pltpu.async_copy(d0_hbm.at[pl.ds(base, TPW)], i0_v.at[0], s0)
        c1 = pltpu.async_copy(d1_hbm.at[pl.ds(base, TPW)], i1_v.at[0], s1)
        c2 = pltpu.async_copy(hid_hbm.at[pl.ds(base, TPW)], rows_v, s2)
        c0.wait()
        c1.wait()
        c2.wait()
        c3 = pltpu.async_copy(rows_v, xpad_hbm.at[i0_v.at[0]], s0)
        c4 = pltpu.async_copy(rows_v, xpad_hbm.at[i1_v.at[0]], s1)
        c3.wait()
        c4.wait()

    return body(hid, d0, d1)


def _wmap(i, m):
    return (jnp.minimum(m[i], NUM_EXPERTS - 1), 0, 0)


def _gmm_body(bmap_ref, x_ref, gw_ref, uw_ref, dw_ref, o_ref):
    @pl.when(bmap_ref[pl.program_id(0)] < NUM_EXPERTS)
    def _():
        x = x_ref[...]
        g = jnp.dot(x, gw_ref[0], preferred_element_type=jnp.float32)
        u = jnp.dot(x, uw_ref[0], preferred_element_type=jnp.float32)
        a = (g / (1.0 + jnp.exp(-g))) * u
        o_ref[...] = jnp.dot(a, dw_ref[0], preferred_element_type=jnp.float32)


def _grouped_ffn(bmap, x_pad, gate_w, up_w, down_w):
    grid_spec = pltpu.PrefetchScalarGridSpec(
        num_scalar_prefetch=1,
        grid=(NUM_BLOCKS,),
        in_specs=[
            pl.BlockSpec((BM, HIDDEN), lambda i, m: (i, 0)),
            pl.BlockSpec((1, HIDDEN, INTER), _wmap),
            pl.BlockSpec((1, HIDDEN, INTER), _wmap),
            pl.BlockSpec((1, INTER, HIDDEN), _wmap),
        ],
        out_specs=pl.BlockSpec((BM, HIDDEN), lambda i, m: (i, 0)),
    )
    return pl.pallas_call(
        _gmm_body,
        grid_spec=grid_spec,
        out_shape=jax.ShapeDtypeStruct((PAD_ROWS, HIDDEN), jnp.float32),
    )(bmap, x_pad, gate_w, up_w, down_w)


def _sc_gather_combine(ypad, d0, d1, w0r, w1r):
    @functools.partial(
        pl.kernel,
        out_type=jax.ShapeDtypeStruct((TOKENS, HIDDEN), jnp.float32),
        mesh=_sc_mesh(),
        scratch_types=[
            pltpu.VMEM((1, TPW), jnp.int32),
            pltpu.VMEM((1, TPW), jnp.int32),
            pltpu.VMEM((TPW, 16), jnp.float32),
            pltpu.VMEM((TPW, 16), jnp.float32),
            pltpu.VMEM((TPW, HIDDEN), jnp.float32),
            pltpu.VMEM((TPW, HIDDEN), jnp.float32),
            pltpu.SemaphoreType.DMA,
            pltpu.SemaphoreType.DMA,
            pltpu.SemaphoreType.DMA,
            pltpu.SemaphoreType.DMA,
        ],
    )
    def body(ypad_hbm, d0_hbm, d1_hbm, w0_hbm, w1_hbm, out_hbm,
             i0_v, i1_v, w0_v, w1_v, r0_v, r1_v, s0, s1, s2, s3):
        wid = jax.lax.axis_index("s") * 2 + jax.lax.axis_index("c")
        base = wid * TPW
        c0 = pltpu.async_copy(d0_hbm.at[pl.ds(base, TPW)], i0_v.at[0], s0)
        c1 = pltpu.async_copy(d1_hbm.at[pl.ds(base, TPW)], i1_v.at[0], s1)
        c2 = pltpu.async_copy(w0_hbm.at[pl.ds(base, TPW)], w0_v, s2)
        c3 = pltpu.async_copy(w1_hbm.at[pl.ds(base, TPW)], w1_v, s3)
        c0.wait()
        c1.wait()
        hw = TPW // 2
        g0a = pltpu.async_copy(ypad_hbm.at[i0_v.at[0, pl.ds(0, hw)]],
                               r0_v.at[pl.ds(0, hw)], s0)
        g1a = pltpu.async_copy(ypad_hbm.at[i1_v.at[0, pl.ds(0, hw)]],
                               r1_v.at[pl.ds(0, hw)], s1)
        g0b = pltpu.async_copy(ypad_hbm.at[i0_v.at[0, pl.ds(hw, hw)]],
                               r0_v.at[pl.ds(hw, hw)], s2)
        g1b = pltpu.async_copy(ypad_hbm.at[i1_v.at[0, pl.ds(hw, hw)]],
                               r1_v.at[pl.ds(hw, hw)], s3)
        c2.wait()
        c3.wait()
        g0a.wait()
        g1a.wait()

        @pl.loop(0, hw)
        def _row_a(j):
            wv0 = w0_v[j]
            wv1 = w1_v[j]
            for c in range(0, HIDDEN, 16):
                s = (j, pl.ds(c, 16))
                r0_v[s] = r0_v[s] * wv0 + r1_v[s] * wv1

        st_a = pltpu.async_copy(r0_v.at[pl.ds(0, hw)],
                                out_hbm.at[pl.ds(base, hw)], s0)
        g0b.wait()
        g1b.wait()

        @pl.loop(hw, TPW)
        def _row_b(j):
            wv0 = w0_v[j]
            wv1 = w1_v[j]
            for c in range(0, HIDDEN, 16):
                s = (j, pl.ds(c, 16))
                r0_v[s] = r0_v[s] * wv0 + r1_v[s] * wv1

        st_a.wait()
        pltpu.sync_copy(r0_v.at[pl.ds(hw, hw)], out_hbm.at[pl.ds(base + hw, hw)])

    return body(ypad, d0, d1, w0r, w1r)


def kernel(hidden_states, router_logits, gate_w, up_w, down_w):
    d0, d1, w0r, w1r, bmap = _routing(router_logits)
    x_pad = _sc_scatter_rows(hidden_states, d0[0], d1[0])
    y_pad = _grouped_ffn(bmap[0], x_pad, gate_w, up_w, down_w)
    return _sc_gather_combine(y_pad, d0[0], d1[0], w0r, w1r)
